# baseline (speedup 1.0000x reference)
import functools

import jax
import jax.numpy as jnp
from jax import lax
from jax.experimental import pallas as pl
from jax.experimental.pallas import tpu as pltpu
from jax.experimental.pallas import tpu_sc as plsc

N = 10000
E = 320000
G = 64
EPS = 1e-5
NEG = -3.4028235e38

R = 2000
NBLK = N // R

NC = 2
NS = 16
NW = NC * NS
CH = 128
NCHUNK = 80
EPAD = NW * NCHUNK * CH
SLICE = 632
NROW = NS * SLICE


HALF = NCHUNK // 2


def _sc_body(z_hbm, srcs_hbm, dsts_hbm, out_hbm, src_v, dst_v, rows, aggsh, sem):
    c = lax.axis_index("c")
    s = lax.axis_index("s")
    wid = c * NS + s

    @pl.loop(0, CH)
    def _zr(r):
        @pl.loop(0, 128, step=16)
        def _zc(k):
            rows.at[0, r, pl.ds(k, 16)][...] = jnp.zeros((16,), jnp.float32)

    @pl.loop(0, 4)
    def _zs(b):
        pltpu.sync_copy(rows.at[0], aggsh.at[pl.ds(s * SLICE + b * CH, CH)])

    pltpu.sync_copy(
        rows.at[0, pl.ds(0, SLICE - 4 * CH)],
        aggsh.at[pl.ds(s * SLICE + 4 * CH, SLICE - 4 * CH)],
    )
    plsc.subcore_barrier()

    for half in range(2):
        pltpu.sync_copy(srcs_hbm.at[wid, pl.ds(half * HALF, HALF)], src_v)
        pltpu.sync_copy(dsts_hbm.at[wid, pl.ds(half * HALF, HALF)], dst_v)
        pltpu.sync_copy(z_hbm.at[src_v.at[0]], rows.at[0])

        @pl.loop(0, HALF - 2, step=2)
        def _mn(j):
            for b in range(2):
                jj = j + b
                nxt = pltpu.async_copy(z_hbm.at[src_v.at[jj + 1]], rows.at[1 - b], sem)
                pltpu.sync_copy(rows.at[b], aggsh.at[dst_v.at[jj]], add=True)
                nxt.wait()

        nxt = pltpu.async_copy(z_hbm.at[src_v.at[HALF - 1]], rows.at[1], sem)
        pltpu.sync_copy(rows.at[0], aggsh.at[dst_v.at[HALF - 2]], add=True)
        nxt.wait()
        pltpu.sync_copy(rows.at[1], aggsh.at[dst_v.at[HALF - 1]], add=True)

    plsc.subcore_barrier()
    pltpu.sync_copy(
        aggsh.at[pl.ds(s * SLICE, SLICE)],
        out_hbm.at[c, pl.ds(s * SLICE, SLICE)],
    )


_sc_scatter = functools.partial(
    pl.kernel,
    out_type=jax.ShapeDtypeStruct((NC, NROW, 128), jnp.float32),
    mesh=plsc.VectorSubcoreMesh(core_axis_name="c", subcore_axis_name="s"),
    scratch_types=[
        pltpu.VMEM((NCHUNK // 2, CH), jnp.int32),
        pltpu.VMEM((NCHUNK // 2, CH), jnp.int32),
        pltpu.VMEM((2, CH, 128), jnp.float32),
        pltpu.VMEM_SHARED((NROW, 128), jnp.float32),
        pltpu.SemaphoreType.DMA,
    ],
)(_sc_body)


def _dot(a, b):
    return jnp.dot(a, b, preferred_element_type=jnp.float32)


def _tc_z_body(x_ref, w_ref, z_ref):
    z_ref[...] = _dot(x_ref[...], w_ref[...])


def _tc_branches_body(x_ref, w1, b1, w2, b2, w3, b3, o_ref):
    x = x_ref[...]
    a = jnp.maximum(_dot(x, w1[...]) + b1[...], 0.0)
    p = jnp.maximum(_dot(x, w2[...]) + b2[...], 0.0)
    q = jnp.maximum(_dot(x, w3[...]) + b3[...], 0.0)
    o_ref[...] = jnp.concatenate([a, p * q], axis=1)


def _tc_combine_body(pre_ref, a_ref, bc, g_ref, bb, m_ref, v_ref, wn, h_ref, z_ref):
    conv = jnp.maximum(a_ref[0, :, :] + a_ref[1, :, :] + bc[...], 0.0)
    pre = pre_ref[...]
    h = jnp.concatenate([pre[:, :128] + conv, pre[:, 128:]], axis=1)
    h = (h - m_ref[...]) / jnp.sqrt(v_ref[...] + EPS) * g_ref[...] + bb[...]
    h_ref[...] = h
    z_ref[...] = _dot(h, wn[...])


def _tc_final_body(pre_ref, a_ref, bt_ref, bc, g_ref, bb, m_ref, v_ref, w2, b2,
                   o_ref, ssum, cnt, mx):
    i = pl.program_id(0)

    @pl.when(i == 0)
    def _():
        ssum[...] = jnp.zeros_like(ssum)
        cnt[...] = jnp.zeros_like(cnt)
        mx[...] = jnp.full_like(mx, NEG)

    conv = jnp.maximum(a_ref[0, :, :] + a_ref[1, :, :] + bc[...], 0.0)
    pre = pre_ref[...]
    h = jnp.concatenate([pre[:, :128] + conv, pre[:, 128:]], axis=1)
    h = (h - m_ref[...]) / jnp.sqrt(v_ref[...] + EPS) * g_ref[...] + bb[...]

    bt = bt_ref[...]
    onehot = (bt == lax.broadcasted_iota(jnp.int32, (1, G), 1)).astype(jnp.float32)
    dn = (((0,), (0,)), ((), ()))
    ssum[...] += lax.dot_general(onehot, h, dn, preferred_element_type=jnp.float32)
    cnt[...] += lax.dot_general(onehot, jnp.ones((R, 192), jnp.float32), dn,
                                preferred_element_type=jnp.float32)

    lo = bt[0, 0]
    hi = bt[R - 1, 0]

    def body(gidx, _):
        mask = bt == gidx
        cand = jnp.max(jnp.where(mask, h, NEG), axis=0, keepdims=True)
        sel = lax.broadcasted_iota(jnp.int32, (G, 1), 0) == gidx
        mx[...] = jnp.where(sel, jnp.maximum(mx[...], cand), mx[...])
        return 0

    lax.fori_loop(lo, hi + 1, body, 0)

    @pl.when(i == NBLK - 1)
    def _():
        cn = cnt[...]
        mean = ssum[...] / jnp.maximum(cn, 1.0)
        mxv = jnp.where(cn > 0.0, mx[...], 0.0)
        pooled = jnp.concatenate([mean, mxv], axis=1)
        logits = _dot(pooled, w2[...]) + b2[...]
        lane = lax.broadcasted_iota(jnp.int32, (G, 128), 1)
        valid = lane < 6
        lm = jnp.max(jnp.where(valid, logits, NEG), axis=1, keepdims=True)
        ex = jnp.where(valid, jnp.exp(logits - lm), 0.0)
        lse = jnp.log(jnp.sum(ex, axis=1, keepdims=True))
        o_ref[...] = logits - lm - lse


def _full(shape):
    return pl.BlockSpec(shape, lambda i: (0,) * len(shape))


_tc_z = pl.pallas_call(
    _tc_z_body,
    grid=(NBLK,),
    in_specs=[pl.BlockSpec((R, 128), lambda i: (i, 0)), _full((128, 128))],
    out_specs=pl.BlockSpec((R, 128), lambda i: (i, 0)),
    out_shape=jax.ShapeDtypeStruct((N, 128), jnp.float32),
)

_tc_branches_l1 = pl.pallas_call(
    _tc_branches_body,
    grid=(NBLK,),
    in_specs=[
        pl.BlockSpec((R, 128), lambda i: (i, 0)),
        _full((128, 128)), _full((1, 128)),
        _full((128, 64)), _full((1, 64)),
        _full((128, 64)), _full((1, 64)),
    ],
    out_specs=pl.BlockSpec((R, 192), lambda i: (i, 0)),
    out_shape=jax.ShapeDtypeStruct((N, 192), jnp.float32),
)

_tc_branches_l2 = pl.pallas_call(
    _tc_branches_body,
    grid=(NBLK,),
    in_specs=[
        pl.BlockSpec((R, 192), lambda i: (i, 0)),
        _full((192, 128)), _full((1, 128)),
        _full((192, 64)), _full((1, 64)),
        _full((192, 64)), _full((1, 64)),
    ],
    out_specs=pl.BlockSpec((R, 192), lambda i: (i, 0)),
    out_shape=jax.ShapeDtypeStruct((N, 192), jnp.float32),
)

_tc_combine = pl.pallas_call(
    _tc_combine_body,
    grid=(NBLK,),
    in_specs=[
        pl.BlockSpec((R, 192), lambda i: (i, 0)),
        pl.BlockSpec((2, R, 128), lambda i: (0, i, 0)),
        _full((1, 128)),
        _full((1, 192)), _full((1, 192)), _full((1, 192)), _full((1, 192)),
        _full((192, 128)),
    ],
    out_specs=[
        pl.BlockSpec((R, 192), lambda i: (i, 0)),
        pl.BlockSpec((R, 128), lambda i: (i, 0)),
    ],
    out_shape=[
        jax.ShapeDtypeStruct((N, 192), jnp.float32),
        jax.ShapeDtypeStruct((N, 128), jnp.float32),
    ],
)

_tc_final = pl.pallas_call(
    _tc_final_body,
    grid=(NBLK,),
    in_specs=[
        pl.BlockSpec((R, 192), lambda i: (i, 0)),
        pl.BlockSpec((2, R, 128), lambda i: (0, i, 0)),
        pl.BlockSpec((R, 1), lambda i: (i, 0)),
        _full((1, 128)),
        _full((1, 192)), _full((1, 192)), _full((1, 192)), _full((1, 192)),
        _full((384, 128)), _full((1, 128)),
    ],
    out_specs=pl.BlockSpec((G, 128), lambda i: (0, 0)),
    out_shape=jax.ShapeDtypeStruct((G, 128), jnp.float32),
    scratch_shapes=[
        pltpu.VMEM((G, 192), jnp.float32),
        pltpu.VMEM((G, 192), jnp.float32),
        pltpu.VMEM((G, 192), jnp.float32),
    ],
)


@jax.jit
def kernel(x, edge_index, batch, W11, b11, W12, b12, W13, b13, Wc1, bc1,
           W21, b21, W22, b22, W23, b23, Wc2, bc2,
           bn1_g, bn1_b, bn1_m, bn1_v, bn2_g, bn2_b, bn2_m, bn2_v, W2, b2):
    src = edge_index[0].astype(jnp.int32)
    dst = edge_index[1].astype(jnp.int32)
    pad = EPAD - E
    pad_dst = N + jnp.arange(pad, dtype=jnp.int32) % (NROW - N)
    srcs = jnp.concatenate([src, jnp.zeros((pad,), jnp.int32)]).reshape(NW, NCHUNK, CH)
    dsts = jnp.concatenate([dst, pad_dst]).reshape(NW, NCHUNK, CH)
    bt = batch.astype(jnp.int32).reshape(N, 1)

    r1 = lambda a: a.reshape(1, -1)
    W2p = jnp.pad(W2, ((0, 0), (0, 128 - W2.shape[1])))
    b2p = jnp.pad(b2, (0, 128 - b2.shape[0])).reshape(1, 128)

    z1 = _tc_z(x, Wc1)
    agg1 = _sc_scatter(z1, srcs, dsts)
    pre1 = _tc_branches_l1(x, W11, r1(b11), W12, r1(b12), W13, r1(b13))
    h1, z2 = _tc_combine(pre1, agg1[:, :N, :], r1(bc1),
                         r1(bn1_g), r1(bn1_b), r1(bn1_m), r1(bn1_v), Wc2)

    agg2 = _sc_scatter(z2, srcs, dsts)
    pre2 = _tc_branches_l2(h1, W21, r1(b21), W22, r1(b22), W23, r1(b23))

    out = _tc_final(pre2, agg2[:, :N, :], bt, r1(bc2),
                    r1(bn2_g), r1(bn2_b), r1(bn2_m), r1(bn2_v), W2p, b2p)
    return out[:, :6]


# speedup vs baseline: 1.0135x; 1.0135x over previous
import functools

import jax
import jax.numpy as jnp
from jax import lax
from jax.experimental import pallas as pl
from jax.experimental.pallas import tpu as pltpu
from jax.experimental.pallas import tpu_sc as plsc

N = 10000
E = 320000
G = 64
EPS = 1e-5
NEG = -3.4028235e38

R = 2000
NBLK = N // R

NC = 2
NS = 16
NW = NC * NS
CH = 128
NCH0 = 128
NCH1 = 32
EPAD = NS * (NCH0 + NCH1) * CH
SLICE = 632
NROW = NS * SLICE


def _sc_body(z_hbm, srcs0_hbm, dsts0_hbm, srcs1_hbm, dsts1_hbm, out_hbm,
             src_v, dst_v, rows, aggsh, sem):
    c = lax.axis_index("c")
    s = lax.axis_index("s")

    @pl.loop(0, CH)
    def _zr(r):
        @pl.loop(0, 128, step=16)
        def _zc(k):
            rows.at[0, r, pl.ds(k, 16)][...] = jnp.zeros((16,), jnp.float32)

    @pl.loop(0, 4)
    def _zs(b):
        pltpu.sync_copy(rows.at[0], aggsh.at[pl.ds(s * SLICE + b * CH, CH)])

    pltpu.sync_copy(
        rows.at[0, pl.ds(0, SLICE - 4 * CH)],
        aggsh.at[pl.ds(s * SLICE + 4 * CH, SLICE - 4 * CH)],
    )
    plsc.subcore_barrier()

    def process(src_h, dst_h, nch):
        half = nch // 2
        for h in range(2):
            pltpu.sync_copy(src_h.at[pl.ds(h * half, half)], src_v.at[pl.ds(0, half)])
            pltpu.sync_copy(dst_h.at[pl.ds(h * half, half)], dst_v.at[pl.ds(0, half)])
            pltpu.sync_copy(z_hbm.at[src_v.at[0]], rows.at[0])

            @pl.loop(0, half - 2, step=2)
            def _mn(j):
                for b in range(2):
                    jj = j + b
                    nxt = pltpu.async_copy(z_hbm.at[src_v.at[jj + 1]], rows.at[1 - b], sem)
                    pltpu.sync_copy(rows.at[b], aggsh.at[dst_v.at[jj]], add=True)
                    nxt.wait()

            nxt = pltpu.async_copy(z_hbm.at[src_v.at[half - 1]], rows.at[1], sem)
            pltpu.sync_copy(rows.at[0], aggsh.at[dst_v.at[half - 2]], add=True)
            nxt.wait()
            pltpu.sync_copy(rows.at[1], aggsh.at[dst_v.at[half - 1]], add=True)

    @pl.when(c == 0)
    def _():
        process(srcs0_hbm.at[s], dsts0_hbm.at[s], NCH0)

    @pl.when(c == 1)
    def _():
        process(srcs1_hbm.at[s], dsts1_hbm.at[s], NCH1)

    plsc.subcore_barrier()
    pltpu.sync_copy(
        aggsh.at[pl.ds(s * SLICE, SLICE)],
        out_hbm.at[c, pl.ds(s * SLICE, SLICE)],
    )


_sc_scatter = functools.partial(
    pl.kernel,
    out_type=jax.ShapeDtypeStruct((NC, NROW, 128), jnp.float32),
    mesh=plsc.VectorSubcoreMesh(core_axis_name="c", subcore_axis_name="s"),
    scratch_types=[
        pltpu.VMEM((NCH0 // 2, CH), jnp.int32),
        pltpu.VMEM((NCH0 // 2, CH), jnp.int32),
        pltpu.VMEM((2, CH, 128), jnp.float32),
        pltpu.VMEM_SHARED((NROW, 128), jnp.float32),
        pltpu.SemaphoreType.DMA,
    ],
)(_sc_body)


def _dot(a, b):
    return jnp.dot(a, b, preferred_element_type=jnp.float32)


def _tc_z_body(x_ref, w_ref, z_ref):
    z_ref[...] = _dot(x_ref[...], w_ref[...])


def _tc_branches_body(x_ref, w1, b1, w2, b2, w3, b3, o_ref):
    x = x_ref[...]
    a = jnp.maximum(_dot(x, w1[...]) + b1[...], 0.0)
    p = jnp.maximum(_dot(x, w2[...]) + b2[...], 0.0)
    q = jnp.maximum(_dot(x, w3[...]) + b3[...], 0.0)
    o_ref[...] = jnp.concatenate([a, p * q], axis=1)


def _tc_combine_body(pre_ref, a_ref, bc, g_ref, bb, m_ref, v_ref, wn, h_ref, z_ref):
    conv = jnp.maximum(a_ref[0, :, :] + a_ref[1, :, :] + bc[...], 0.0)
    pre = pre_ref[...]
    h = jnp.concatenate([pre[:, :128] + conv, pre[:, 128:]], axis=1)
    h = (h - m_ref[...]) / jnp.sqrt(v_ref[...] + EPS) * g_ref[...] + bb[...]
    h_ref[...] = h
    z_ref[...] = _dot(h, wn[...])


def _tc_final_body(pre_ref, a_ref, bt_ref, bc, g_ref, bb, m_ref, v_ref, w2, b2,
                   o_ref, ssum, cnt, mx):
    i = pl.program_id(0)

    @pl.when(i == 0)
    def _():
        ssum[...] = jnp.zeros_like(ssum)
        cnt[...] = jnp.zeros_like(cnt)
        mx[...] = jnp.full_like(mx, NEG)

    conv = jnp.maximum(a_ref[0, :, :] + a_ref[1, :, :] + bc[...], 0.0)
    pre = pre_ref[...]
    h = jnp.concatenate([pre[:, :128] + conv, pre[:, 128:]], axis=1)
    h = (h - m_ref[...]) / jnp.sqrt(v_ref[...] + EPS) * g_ref[...] + bb[...]

    bt = bt_ref[...]
    onehot = (bt == lax.broadcasted_iota(jnp.int32, (1, G), 1)).astype(jnp.float32)
    dn = (((0,), (0,)), ((), ()))
    ssum[...] += lax.dot_general(onehot, h, dn, preferred_element_type=jnp.float32)
    cnt[...] += lax.dot_general(onehot, jnp.ones((R, 192), jnp.float32), dn,
                                preferred_element_type=jnp.float32)

    lo = bt[0, 0]
    hi = bt[R - 1, 0]

    def body(gidx, _):
        mask = bt == gidx
        cand = jnp.max(jnp.where(mask, h, NEG), axis=0, keepdims=True)
        sel = lax.broadcasted_iota(jnp.int32, (G, 1), 0) == gidx
        mx[...] = jnp.where(sel, jnp.maximum(mx[...], cand), mx[...])
        return 0

    lax.fori_loop(lo, hi + 1, body, 0)

    @pl.when(i == NBLK - 1)
    def _():
        cn = cnt[...]
        mean = ssum[...] / jnp.maximum(cn, 1.0)
        mxv = jnp.where(cn > 0.0, mx[...], 0.0)
        pooled = jnp.concatenate([mean, mxv], axis=1)
        logits = _dot(pooled, w2[...]) + b2[...]
        lane = lax.broadcasted_iota(jnp.int32, (G, 128), 1)
        valid = lane < 6
        lm = jnp.max(jnp.where(valid, logits, NEG), axis=1, keepdims=True)
        ex = jnp.where(valid, jnp.exp(logits - lm), 0.0)
        lse = jnp.log(jnp.sum(ex, axis=1, keepdims=True))
        o_ref[...] = logits - lm - lse


def _full(shape):
    return pl.BlockSpec(shape, lambda i: (0,) * len(shape))


_tc_z = pl.pallas_call(
    _tc_z_body,
    grid=(NBLK,),
    in_specs=[pl.BlockSpec((R, 128), lambda i: (i, 0)), _full((128, 128))],
    out_specs=pl.BlockSpec((R, 128), lambda i: (i, 0)),
    out_shape=jax.ShapeDtypeStruct((N, 128), jnp.float32),
)

_tc_branches_l1 = pl.pallas_call(
    _tc_branches_body,
    grid=(NBLK,),
    in_specs=[
        pl.BlockSpec((R, 128), lambda i: (i, 0)),
        _full((128, 128)), _full((1, 128)),
        _full((128, 64)), _full((1, 64)),
        _full((128, 64)), _full((1, 64)),
    ],
    out_specs=pl.BlockSpec((R, 192), lambda i: (i, 0)),
    out_shape=jax.ShapeDtypeStruct((N, 192), jnp.float32),
)

_tc_branches_l2 = pl.pallas_call(
    _tc_branches_body,
    grid=(NBLK,),
    in_specs=[
        pl.BlockSpec((R, 192), lambda i: (i, 0)),
        _full((192, 128)), _full((1, 128)),
        _full((192, 64)), _full((1, 64)),
        _full((192, 64)), _full((1, 64)),
    ],
    out_specs=pl.BlockSpec((R, 192), lambda i: (i, 0)),
    out_shape=jax.ShapeDtypeStruct((N, 192), jnp.float32),
)

_tc_combine = pl.pallas_call(
    _tc_combine_body,
    grid=(NBLK,),
    in_specs=[
        pl.BlockSpec((R, 192), lambda i: (i, 0)),
        pl.BlockSpec((2, R, 128), lambda i: (0, i, 0)),
        _full((1, 128)),
        _full((1, 192)), _full((1, 192)), _full((1, 192)), _full((1, 192)),
        _full((192, 128)),
    ],
    out_specs=[
        pl.BlockSpec((R, 192), lambda i: (i, 0)),
        pl.BlockSpec((R, 128), lambda i: (i, 0)),
    ],
    out_shape=[
        jax.ShapeDtypeStruct((N, 192), jnp.float32),
        jax.ShapeDtypeStruct((N, 128), jnp.float32),
    ],
)

_tc_final = pl.pallas_call(
    _tc_final_body,
    grid=(NBLK,),
    in_specs=[
        pl.BlockSpec((R, 192), lambda i: (i, 0)),
        pl.BlockSpec((2, R, 128), lambda i: (0, i, 0)),
        pl.BlockSpec((R, 1), lambda i: (i, 0)),
        _full((1, 128)),
        _full((1, 192)), _full((1, 192)), _full((1, 192)), _full((1, 192)),
        _full((384, 128)), _full((1, 128)),
    ],
    out_specs=pl.BlockSpec((G, 128), lambda i: (0, 0)),
    out_shape=jax.ShapeDtypeStruct((G, 128), jnp.float32),
    scratch_shapes=[
        pltpu.VMEM((G, 192), jnp.float32),
        pltpu.VMEM((G, 192), jnp.float32),
        pltpu.VMEM((G, 192), jnp.float32),
    ],
)


@jax.jit
def kernel(x, edge_index, batch, W11, b11, W12, b12, W13, b13, Wc1, bc1,
           W21, b21, W22, b22, W23, b23, Wc2, bc2,
           bn1_g, bn1_b, bn1_m, bn1_v, bn2_g, bn2_b, bn2_m, bn2_v, W2, b2):
    src = edge_index[0].astype(jnp.int32)
    dst = edge_index[1].astype(jnp.int32)
    pad = EPAD - E
    pad_dst = N + jnp.arange(pad, dtype=jnp.int32) % (NROW - N)
    src_p = jnp.concatenate([src, jnp.zeros((pad,), jnp.int32)])
    dst_p = jnp.concatenate([dst, pad_dst])
    e0 = NS * NCH0 * CH
    srcs0 = src_p[:e0].reshape(NS, NCH0, CH)
    dsts0 = dst_p[:e0].reshape(NS, NCH0, CH)
    srcs1 = src_p[e0:].reshape(NS, NCH1, CH)
    dsts1 = dst_p[e0:].reshape(NS, NCH1, CH)
    bt = batch.astype(jnp.int32).reshape(N, 1)

    r1 = lambda a: a.reshape(1, -1)
    W2p = jnp.pad(W2, ((0, 0), (0, 128 - W2.shape[1])))
    b2p = jnp.pad(b2, (0, 128 - b2.shape[0])).reshape(1, 128)

    z1 = _tc_z(x, Wc1)
    agg1 = _sc_scatter(z1, srcs0, dsts0, srcs1, dsts1)
    pre1 = _tc_branches_l1(x, W11, r1(b11), W12, r1(b12), W13, r1(b13))
    h1, z2 = _tc_combine(pre1, agg1[:, :N, :], r1(bc1),
                         r1(bn1_g), r1(bn1_b), r1(bn1_m), r1(bn1_v), Wc2)

    agg2 = _sc_scatter(z2, srcs0, dsts0, srcs1, dsts1)
    pre2 = _tc_branches_l2(h1, W21, r1(b21), W22, r1(b22), W23, r1(b23))

    out = _tc_final(pre2, agg2[:, :N, :], bt, r1(bc2),
                    r1(bn2_g), r1(bn2_b), r1(bn2_m), r1(bn2_v), W2p, b2p)
    return out[:, :6]
